# baseline (speedup 1.0000x reference)
import jax
import jax.numpy as jnp
from jax import lax
from jax.experimental import pallas as pl
from jax.experimental.pallas import tpu as pltpu
from jax.experimental.pallas import tpu_sc as plsc

_G, _N, _F = 8, 512, 128
_H = 128
_D_TP = 64
_N_MLP = 256
_N_GCN = 4
_B = 64
_GPB = 2
_STEPS = _G // _GPB
_L = 16
_NCHUNK = _N // _L


def _softmax(z):
    z = z - jnp.max(z, axis=-1, keepdims=True)
    e = jnp.exp(z)
    return e / jnp.sum(e, axis=-1, keepdims=True)


def _gcn_body(gx_ref, ga_ref, W_in_ref, b_in_ref, W_gcn_ref, b_gcn_ref,
              xf_ref):
    def bdot(a, b):
        return jnp.dot(a.astype(jnp.bfloat16), b.astype(jnp.bfloat16),
                       preferred_element_type=jnp.float32)

    for j in range(_GPB):
        gx = gx_ref[j]
        ga = ga_ref[j].astype(jnp.bfloat16)
        x = bdot(gx, W_in_ref[...])
        x = jnp.maximum(x + b_in_ref[...], 0.0)
        to_add = x
        for i in range(_N_GCN):
            z = jnp.dot(ga, x.astype(jnp.bfloat16),
                        preferred_element_type=jnp.float32)
            z = bdot(z, W_gcn_ref[i])
            z = z + b_gcn_ref[i]
            if i < _N_GCN - 1:
                x = jnp.maximum(z, 0.0)
            else:
                x = _softmax(z)
        xf_ref[j] = x + to_add


def _run_gcn(graph_xs_all, graph_as_all, W_in, b_in, W_gcn, b_gcn):
    full = lambda shape: pl.BlockSpec(shape, lambda g: (0,) * len(shape))
    return pl.pallas_call(
        _gcn_body,
        grid=(_STEPS,),
        in_specs=[
            pl.BlockSpec((_GPB, _N, _F), lambda g: (g, 0, 0)),
            pl.BlockSpec((_GPB, _N, _N), lambda g: (g, 0, 0)),
            full((_F, _H)), full((_H,)),
            full((_N_GCN, _H, _H)), full((_N_GCN, _H)),
        ],
        out_specs=pl.BlockSpec((_GPB, _N, _H), lambda g: (g, 0, 0)),
        out_shape=jax.ShapeDtypeStruct((_G, _N, _H), jnp.float32),
    )(graph_xs_all, graph_as_all, W_in, b_in, W_gcn, b_gcn)


def _pool_body(xf_hbm, mask_hbm, idx_hbm, cov_hbm,
               idx_v, mask_v, idxbuf, rows_v, cov_v, sem):
    info = plsc.get_sparse_core_info()
    nc = info.num_cores
    wid = lax.axis_index("s") * nc + lax.axis_index("c")

    pltpu.sync_copy(idx_hbm, idx_v)
    lanes = lax.broadcasted_iota(jnp.int32, (_L,), 0)

    for j in range(_B // 32):
        b = wid * (_B // 32) + j
        g_vec = plsc.load_gather(idx_v, [jnp.full((_L,), b, jnp.int32)])
        base_vec = g_vec * _N

        pltpu.sync_copy(mask_hbm.at[b], mask_v)

        for k in range(_NCHUNK):
            idxbuf[pl.ds(k * _L, _L)] = base_vec

        def compact(c, cnt):
            mv = mask_v[pl.ds(c * _L, _L)]
            msk = mv > 0.0
            mi = jnp.where(msk, 1, 0).astype(jnp.int32)
            pos = jnp.cumsum(mi) - 1
            glob = base_vec + c * _L + lanes
            plsc.store_scatter(idxbuf, [pos + cnt], glob, mask=msk)
            return cnt + jnp.sum(mi)

        cnt = lax.fori_loop(0, _NCHUNK, compact, jnp.int32(0))
        nch = (cnt + _L - 1) // _L

        def gather_step(t, acc):
            iv = idxbuf[pl.ds(t * _L, _L)]
            pltpu.async_copy(xf_hbm.at[iv], rows_v, sem).wait()
            out = []
            for s in range(_H // _L):
                seg = jnp.zeros((_L,), jnp.float32)
                for r in range(_L):
                    valid = (t * _L + r) < cnt
                    row = rows_v[r, pl.ds(s * _L, _L)]
                    seg = seg + jnp.where(valid, row, 0.0)
                out.append(acc[s] + seg)
            return tuple(out)

        acc0 = tuple(jnp.zeros((_L,), jnp.float32) for _ in range(_H // _L))
        acc = lax.fori_loop(0, nch, gather_step, acc0)

        denom_vec = jnp.maximum(jnp.full((_L,), cnt.astype(jnp.float32)), 1.0)
        scale = 1.0 / denom_vec
        for s in range(_H // _L):
            cov_v[pl.ds(s * _L, _L)] = acc[s] * scale
        pltpu.sync_copy(cov_v, cov_hbm.at[b])


def _run_pool(xf_flat, mask_f, idx):
    mesh = plsc.VectorSubcoreMesh(core_axis_name="c", subcore_axis_name="s")
    return pl.kernel(
        _pool_body,
        out_type=jax.ShapeDtypeStruct((_B, _H), jnp.float32),
        mesh=mesh,
        compiler_params=pltpu.CompilerParams(needs_layout_passes=False),
        scratch_types=[
            pltpu.VMEM((_B,), jnp.int32),
            pltpu.VMEM((_N,), jnp.float32),
            pltpu.VMEM((_N,), jnp.int32),
            pltpu.VMEM((_L, _H), jnp.float32),
            pltpu.VMEM((_H,), jnp.float32),
            pltpu.SemaphoreType.DMA,
        ],
    )(xf_flat, mask_f, idx)


def _head_body(cov_ref, tp_ref, W_tp1_ref, b_tp1_ref, W_tp2_ref, b_tp2_ref,
               W_f1_ref, b_f1_ref, W_f2_ref, b_f2_ref, out_ref):
    t = jnp.dot(tp_ref[...], W_tp1_ref[...],
                preferred_element_type=jnp.float32) + b_tp1_ref[...]
    t = jnp.maximum(t, 0.0)
    t = jnp.dot(t, W_tp2_ref[...],
                preferred_element_type=jnp.float32) + b_tp2_ref[...]
    tp_e = _softmax(t)
    h = (jnp.dot(cov_ref[...], W_f1_ref[:_H],
                 preferred_element_type=jnp.float32)
         + jnp.dot(tp_e, W_f1_ref[_H:],
                   preferred_element_type=jnp.float32)
         + b_f1_ref[...])
    h = jnp.maximum(h, 0.0)
    o = jnp.dot(h, W_f2_ref[...],
                preferred_element_type=jnp.float32) + b_f2_ref[...]
    out_ref[...] = 1.0 / (1.0 + jnp.exp(-o))


def _run_head(cov, test_parameters, W_tp1, b_tp1, W_tp2, b_tp2,
              W_f1, b_f1, W_f2, b_f2):
    return pl.pallas_call(
        _head_body,
        out_shape=jax.ShapeDtypeStruct((_B, 1), jnp.float32),
    )(cov, test_parameters, W_tp1, b_tp1, W_tp2, b_tp2,
      W_f1, b_f1, W_f2, b_f2)


def kernel(test_parameters, graph, coverpoint_mask, graph_xs_all, graph_as_all,
           W_in, b_in, W_gcn, b_gcn, W_tp1, b_tp1, W_tp2, b_tp2,
           W_f1, b_f1, W_f2, b_f2):
    idx = graph[:, 0].astype(jnp.int32)
    mask_f = coverpoint_mask.astype(jnp.float32)

    xf = _run_gcn(graph_xs_all, graph_as_all, W_in, b_in, W_gcn, b_gcn)
    cov = _run_pool(xf.reshape(_G * _N, _H), mask_f, idx)
    out = _run_head(cov, test_parameters, W_tp1, b_tp1, W_tp2, b_tp2,
                    W_f1, b_f1, W_f2, b_f2)
    return out
